# baseline (speedup 1.0000x reference)
import functools

import jax
import jax.numpy as jnp
from jax import lax
from jax.experimental import pallas as pl
from jax.experimental.pallas import tpu as pltpu
from jax.experimental.pallas import tpu_sc as plsc

KNN = 10
EPS = 1e-5
BIGF = 3.0e38
CT11 = (((1,), (1,)), ((), ()))
HI = jax.lax.Precision.HIGHEST


def _lrelu(x):
    return jnp.where(x > 0, x, 0.2 * x)


def _topk_body(n_total, tq, pts_ref, cf_ref, x_ref, w_ref,
               idx_ref, u_ref, v_ref):
    b = pl.program_id(0)
    t = pl.program_id(1)
    x = x_ref[0]
    w = w_ref[...]
    ci = w.shape[1] // 2
    wa = w[:, :ci] - w[:, ci:]
    wb = w[:, ci:]
    u_ref[0] = lax.dot_general(x, wa, CT11,
                               preferred_element_type=jnp.float32, precision=HI)
    v_ref[...] = lax.dot_general(x, wb, CT11,
                                 preferred_element_type=jnp.float32, precision=HI)
    a = pts_ref[0]
    c = cf_ref[0]
    g = lax.dot_general(a, c, (((1,), (0,)), ((), ())),
                        preferred_element_type=jnp.float32,
                        precision=lax.Precision.DEFAULT)
    n2r = jnp.sum(a * a, axis=1, keepdims=True)
    n2c = jnp.sum(c * c, axis=0, keepdims=True)
    d = jnp.maximum((-2.0 * g + n2r) + n2c, 1e-12)
    colid = lax.broadcasted_iota(jnp.int32, d.shape, 1)
    base = b * n_total
    for kk in range(KNN + 1):
        rowmin = jnp.min(d, axis=1, keepdims=True)
        amin = jnp.min(jnp.where(d == rowmin, colid, n_total),
                       axis=1, keepdims=True)
        if kk > 0:
            idx_ref[0, :, kk - 1:kk] = amin + base
        if kk <= KNN - 1:
            d = jnp.where(colid == amin, BIGF, d)


def _topk_call(pts8, cf8, ftT, w1):
    B, N, _ = pts8.shape
    Cin = ftT.shape[2]
    Cout = w1.shape[0]
    TQ = 256
    NT = N // TQ
    grid = (B, NT)
    return pl.pallas_call(
        functools.partial(_topk_body, N, TQ),
        grid=grid,
        in_specs=[
            pl.BlockSpec((1, TQ, 8), lambda b, t: (b, t, 0)),
            pl.BlockSpec((1, 8, N), lambda b, t: (b, 0, 0)),
            pl.BlockSpec((1, TQ, Cin), lambda b, t: (b, t, 0)),
            pl.BlockSpec(w1.shape, lambda b, t: (0, 0)),
        ],
        out_specs=[
            pl.BlockSpec((1, TQ, 16), lambda b, t: (b, t, 0)),
            pl.BlockSpec((1, TQ, Cout), lambda b, t: (b, t, 0)),
            pl.BlockSpec((TQ, Cout), lambda b, t: (b * NT + t, 0)),
        ],
        out_shape=[
            jax.ShapeDtypeStruct((B, N, 16), jnp.int32),
            jax.ShapeDtypeStruct((B, N, Cout), jnp.float32),
            jax.ShapeDtypeStruct((B * N, Cout), jnp.float32),
        ],
    )(pts8, cf8, ftT, w1)


def _sc_gather(table, gidx):
    R, D = table.shape
    M = gidx.shape[0]
    NW = 32
    b_per_w = M // NW
    CH = min(b_per_w, 65536 // D)
    n_chunks = b_per_w // CH
    mesh = plsc.VectorSubcoreMesh(core_axis_name="c", subcore_axis_name="s")

    @functools.partial(
        pl.kernel, mesh=mesh,
        out_type=jax.ShapeDtypeStruct((M, D), jnp.float32),
        scratch_types=[
            pltpu.VMEM((CH,), jnp.int32),
            pltpu.VMEM((CH, D), jnp.float32),
            pltpu.SemaphoreType.DMA,
        ],
    )
    def k(table_hbm, idx_hbm, out_hbm, idx_c, rows_v, sem):
        wid = lax.axis_index("s") * 2 + lax.axis_index("c")
        base = wid * b_per_w

        @pl.loop(0, n_chunks)
        def _(j):
            off = base + j * CH
            pltpu.sync_copy(idx_hbm.at[pl.ds(off, CH)], idx_c)
            pltpu.async_copy(table_hbm.at[idx_c], rows_v, sem).wait()
            pltpu.sync_copy(rows_v, out_hbm.at[pl.ds(off, CH)])

    return k(table, gidx)


def _red_body(cw, g_ref, u_ref, pre_ref, st_ref):
    t = pl.program_id(1)
    g0 = g_ref[0, :, 0:cw]
    mx = g0
    sm = g0
    s2 = g0 * g0
    for kk in range(1, KNN):
        gk = g_ref[0, :, kk * cw:(kk + 1) * cw]
        mx = jnp.maximum(mx, gk)
        sm = sm + gk
        s2 = s2 + gk * gk
    u = u_ref[0]
    pre_ref[0] = u + mx
    su = jnp.sum(u, axis=0, keepdims=True)
    su2 = jnp.sum(u * u, axis=0, keepdims=True)
    sv = jnp.sum(sm, axis=0, keepdims=True)
    sv2 = jnp.sum(s2, axis=0, keepdims=True)
    cx = jnp.sum(u * sm, axis=0, keepdims=True)
    z = jnp.zeros_like(su)
    st = jnp.concatenate([su, su2, sv, sv2, cx, z, z, z], axis=0)

    @pl.when(t == 0)
    def _():
        st_ref[0] = st

    @pl.when(t > 0)
    def _():
        st_ref[0] = st_ref[0] + st


def _red_call(g, u):
    B, N, Cw = u.shape
    TB = 512
    grid = (B, N // TB)
    return pl.pallas_call(
        functools.partial(_red_body, Cw),
        grid=grid,
        in_specs=[
            pl.BlockSpec((1, TB, KNN * Cw), lambda b, t: (b, t, 0)),
            pl.BlockSpec((1, TB, Cw), lambda b, t: (b, t, 0)),
        ],
        out_specs=[
            pl.BlockSpec((1, TB, Cw), lambda b, t: (b, t, 0)),
            pl.BlockSpec((1, 8, Cw), lambda b, t: (b, 0, 0)),
        ],
        out_shape=[
            jax.ShapeDtypeStruct((B, N, Cw), jnp.float32),
            jax.ShapeDtypeStruct((B, 8, Cw), jnp.float32),
        ],
    )(g, u)


def _edge_stats(st_ref, n_total):
    st = st_ref[0]
    su, su2, sv, sv2, cx = (st[0:1], st[1:2], st[2:3], st[3:4], st[4:5])
    inv = 1.0 / (n_total * KNN)
    mean = (KNN * su + sv) * inv
    ex2 = (KNN * su2 + 2.0 * cx + sv2) * inv
    var = ex2 - mean * mean
    return mean, lax.rsqrt(var + EPS)


def _norm_uv_body(n_total, pre_ref, st_ref, w_ref, x_ref, u_ref, v_ref):
    mean, rstd = _edge_stats(st_ref, n_total)
    x = _lrelu((pre_ref[0] - mean) * rstd)
    x_ref[0] = x
    w = w_ref[...]
    ci = w.shape[1] // 2
    wa = w[:, :ci] - w[:, ci:]
    wb = w[:, ci:]
    u_ref[0] = lax.dot_general(x, wa, CT11,
                               preferred_element_type=jnp.float32, precision=HI)
    v_ref[...] = lax.dot_general(x, wb, CT11,
                                 preferred_element_type=jnp.float32, precision=HI)


def _norm_uv_call(pre, st, w):
    B, N, Cin = pre.shape
    Cout = w.shape[0]
    TB = 512
    NT = N // TB
    grid = (B, NT)
    return pl.pallas_call(
        functools.partial(_norm_uv_body, N),
        grid=grid,
        in_specs=[
            pl.BlockSpec((1, TB, Cin), lambda b, t: (b, t, 0)),
            pl.BlockSpec((1, 8, Cin), lambda b, t: (b, 0, 0)),
            pl.BlockSpec(w.shape, lambda b, t: (0, 0)),
        ],
        out_specs=[
            pl.BlockSpec((1, TB, Cin), lambda b, t: (b, t, 0)),
            pl.BlockSpec((1, TB, Cout), lambda b, t: (b, t, 0)),
            pl.BlockSpec((TB, Cout), lambda b, t: (b * NT + t, 0)),
        ],
        out_shape=[
            jax.ShapeDtypeStruct((B, N, Cin), jnp.float32),
            jax.ShapeDtypeStruct((B, N, Cout), jnp.float32),
            jax.ShapeDtypeStruct((B * N, Cout), jnp.float32),
        ],
    )(pre, st, w)


def _norm_conv3_body(n_total, c, pre_ref, st_ref, ft_ref, x1_ref, w_ref,
                     y_ref, st3_ref):
    t = pl.program_id(1)
    mean, rstd = _edge_stats(st_ref, n_total)
    x2 = _lrelu((pre_ref[0] - mean) * rstd)
    w = w_ref[...]
    y = lax.dot_general(ft_ref[0], w[:, :c], CT11,
                        preferred_element_type=jnp.float32, precision=HI)
    y = y + lax.dot_general(x1_ref[0], w[:, c:2 * c], CT11,
                            preferred_element_type=jnp.float32, precision=HI)
    y = y + lax.dot_general(x2, w[:, 2 * c:], CT11,
                            preferred_element_type=jnp.float32, precision=HI)
    y_ref[0] = y
    sy = jnp.sum(y, axis=0, keepdims=True)
    sy2 = jnp.sum(y * y, axis=0, keepdims=True)
    z = jnp.zeros_like(sy)
    st = jnp.concatenate([sy, sy2, z, z, z, z, z, z], axis=0)

    @pl.when(t == 0)
    def _():
        st3_ref[0] = st

    @pl.when(t > 0)
    def _():
        st3_ref[0] = st3_ref[0] + st


def _norm_conv3_call(pre2, st2, ftT, x1, w3):
    B, N, C2 = pre2.shape
    C = ftT.shape[2]
    TB = 512
    grid = (B, N // TB)
    return pl.pallas_call(
        functools.partial(_norm_conv3_body, N, C),
        grid=grid,
        in_specs=[
            pl.BlockSpec((1, TB, C2), lambda b, t: (b, t, 0)),
            pl.BlockSpec((1, 8, C2), lambda b, t: (b, 0, 0)),
            pl.BlockSpec((1, TB, C), lambda b, t: (b, t, 0)),
            pl.BlockSpec((1, TB, C), lambda b, t: (b, t, 0)),
            pl.BlockSpec(w3.shape, lambda b, t: (0, 0)),
        ],
        out_specs=[
            pl.BlockSpec((1, TB, C), lambda b, t: (b, t, 0)),
            pl.BlockSpec((1, 8, C), lambda b, t: (b, 0, 0)),
        ],
        out_shape=[
            jax.ShapeDtypeStruct((B, N, C), jnp.float32),
            jax.ShapeDtypeStruct((B, 8, C), jnp.float32),
        ],
    )(pre2, st2, ftT, x1, w3)


def _final_body(n_total, y_ref, st_ref, o_ref):
    st = st_ref[0]
    mean = st[0:1] / n_total
    var = st[1:2] / n_total - mean * mean
    o_ref[0] = _lrelu((y_ref[0] - mean) * lax.rsqrt(var + EPS))


def _final_call(y, st):
    B, N, C = y.shape
    TB = 512
    grid = (B, N // TB)
    return pl.pallas_call(
        functools.partial(_final_body, N),
        grid=grid,
        in_specs=[
            pl.BlockSpec((1, TB, C), lambda b, t: (b, t, 0)),
            pl.BlockSpec((1, 8, C), lambda b, t: (b, 0, 0)),
        ],
        out_specs=pl.BlockSpec((1, TB, C), lambda b, t: (b, t, 0)),
        out_shape=jax.ShapeDtypeStruct((B, N, C), jnp.float32),
    )(y, st)


def kernel(coords, features, W1, W2, W3):
    B, C, N = features.shape
    pts = jnp.transpose(coords, (0, 2, 1))
    pts8 = jnp.concatenate(
        [pts, jnp.zeros((B, N, 5), jnp.float32)], axis=2)
    cf8 = jnp.concatenate(
        [coords, jnp.zeros((B, 5, N), jnp.float32)], axis=1)
    ftT = jnp.transpose(features, (0, 2, 1))

    idx16, u1, v1 = _topk_call(pts8, cf8, ftT, W1)
    gidx = idx16[:, :, :KNN].reshape(-1)
    g1 = _sc_gather(v1, gidx).reshape(B, N, KNN * C)
    pre1, st1 = _red_call(g1, u1)
    x1, u2, v2 = _norm_uv_call(pre1, st1, W2)
    g2 = _sc_gather(v2, gidx).reshape(B, N, KNN * 2 * C)
    pre2, st2 = _red_call(g2, u2)
    y3, st3 = _norm_conv3_call(pre2, st2, ftT, x1, W3)
    out = _final_call(y3, st3)
    return jnp.transpose(out, (0, 2, 1))


# speedup vs baseline: 1.2031x; 1.2031x over previous
import functools

import jax
import jax.numpy as jnp
from jax import lax
from jax.experimental import pallas as pl
from jax.experimental.pallas import tpu as pltpu
from jax.experimental.pallas import tpu_sc as plsc

KNN = 10
EPS = 1e-5
BIGF = 3.0e38
CT11 = (((1,), (1,)), ((), ()))
HI = jax.lax.Precision.HIGHEST


def _lrelu(x):
    return jnp.where(x > 0, x, 0.2 * x)


def _topk_body(n_total, tq, pts_ref, cf_ref, x_ref, w_ref,
               idx_ref, u_ref, v_ref):
    b = pl.program_id(0)
    t = pl.program_id(1)
    x = x_ref[0]
    w = w_ref[...]
    ci = w.shape[1] // 2
    wa = w[:, :ci] - w[:, ci:]
    wb = w[:, ci:]
    u_ref[0] = lax.dot_general(x, wa, CT11,
                               preferred_element_type=jnp.float32, precision=HI)
    v_ref[...] = lax.dot_general(x, wb, CT11,
                                 preferred_element_type=jnp.float32, precision=HI)
    a = pts_ref[0]
    c = cf_ref[0]
    g = lax.dot_general(a, c, (((1,), (0,)), ((), ())),
                        preferred_element_type=jnp.float32,
                        precision=lax.Precision.DEFAULT)
    n2r = jnp.sum(a * a, axis=1, keepdims=True)
    n2c = jnp.sum(c * c, axis=0, keepdims=True)
    d = jnp.maximum((-2.0 * g + n2r) + n2c, 1e-12)
    colid = lax.broadcasted_iota(jnp.int32, d.shape, 1)
    base = b * n_total
    for kk in range(KNN + 1):
        rowmin = jnp.min(d, axis=1, keepdims=True)
        amin = jnp.min(jnp.where(d == rowmin, colid, n_total),
                       axis=1, keepdims=True)
        if kk > 0:
            idx_ref[0, :, kk - 1:kk] = amin + base
        if kk <= KNN - 1:
            d = jnp.where(colid == amin, BIGF, d)


def _topk_call(pts8, cf8, ftT, w1):
    B, N, _ = pts8.shape
    Cin = ftT.shape[2]
    Cout = w1.shape[0]
    TQ = 256
    NT = N // TQ
    grid = (B, NT)
    return pl.pallas_call(
        functools.partial(_topk_body, N, TQ),
        grid=grid,
        in_specs=[
            pl.BlockSpec((1, TQ, 8), lambda b, t: (b, t, 0)),
            pl.BlockSpec((1, 8, N), lambda b, t: (b, 0, 0)),
            pl.BlockSpec((1, TQ, Cin), lambda b, t: (b, t, 0)),
            pl.BlockSpec(w1.shape, lambda b, t: (0, 0)),
        ],
        out_specs=[
            pl.BlockSpec((1, TQ, 16), lambda b, t: (b, t, 0)),
            pl.BlockSpec((1, TQ, Cout), lambda b, t: (b, t, 0)),
            pl.BlockSpec((TQ, Cout), lambda b, t: (b * NT + t, 0)),
        ],
        out_shape=[
            jax.ShapeDtypeStruct((B, N, 16), jnp.int32),
            jax.ShapeDtypeStruct((B, N, Cout), jnp.float32),
            jax.ShapeDtypeStruct((B * N, Cout), jnp.float32),
        ],
    )(pts8, cf8, ftT, w1)


def _sc_gather(table, gidx):
    R, D = table.shape
    M = gidx.shape[0]
    NW = 32
    b_per_w = M // NW
    CH = 32768 // D
    n_pairs = b_per_w // (2 * CH)
    mesh = plsc.VectorSubcoreMesh(core_axis_name="c", subcore_axis_name="s")

    @functools.partial(
        pl.kernel, mesh=mesh,
        out_type=jax.ShapeDtypeStruct((M, D), jnp.float32),
        scratch_types=[
            pltpu.VMEM((b_per_w,), jnp.int32),
            pltpu.VMEM((CH, D), jnp.float32),
            pltpu.VMEM((CH, D), jnp.float32),
            pltpu.SemaphoreType.DMA,
            pltpu.SemaphoreType.DMA,
            pltpu.SemaphoreType.DMA,
        ],
    )
    def k(table_hbm, idx_hbm, out_hbm, idx_all, r0, r1, sg, s0, s1):
        wid = lax.axis_index("s") * 2 + lax.axis_index("c")
        base = wid * b_per_w
        pltpu.sync_copy(idx_hbm.at[pl.ds(base, b_per_w)], idx_all)

        @pl.loop(0, n_pairs)
        def _(t):
            o0 = 2 * t * CH
            o1 = o0 + CH

            @pl.when(t > 0)
            def _():
                pltpu.make_async_copy(
                    r0, out_hbm.at[pl.ds(base + o0 - 2 * CH, CH)], s0).wait()

            pltpu.async_copy(
                table_hbm.at[idx_all.at[pl.ds(o0, CH)]], r0, sg).wait()
            pltpu.async_copy(r0, out_hbm.at[pl.ds(base + o0, CH)], s0)

            @pl.when(t > 0)
            def _():
                pltpu.make_async_copy(
                    r1, out_hbm.at[pl.ds(base + o1 - 2 * CH, CH)], s1).wait()

            pltpu.async_copy(
                table_hbm.at[idx_all.at[pl.ds(o1, CH)]], r1, sg).wait()
            pltpu.async_copy(r1, out_hbm.at[pl.ds(base + o1, CH)], s1)

        last0 = b_per_w - 2 * CH
        pltpu.make_async_copy(
            r0, out_hbm.at[pl.ds(base + last0, CH)], s0).wait()
        pltpu.make_async_copy(
            r1, out_hbm.at[pl.ds(base + last0 + CH, CH)], s1).wait()

    return k(table, gidx)


def _red_body(cw, g_ref, u_ref, pre_ref, st_ref):
    t = pl.program_id(1)
    g0 = g_ref[0]
    mx = g0
    sm = g0
    s2 = g0 * g0
    for kk in range(1, KNN):
        gk = g_ref[kk]
        mx = jnp.maximum(mx, gk)
        sm = sm + gk
        s2 = s2 + gk * gk
    u = u_ref[0]
    pre_ref[0] = u + mx
    su = jnp.sum(u, axis=0, keepdims=True)
    su2 = jnp.sum(u * u, axis=0, keepdims=True)
    sv = jnp.sum(sm, axis=0, keepdims=True)
    sv2 = jnp.sum(s2, axis=0, keepdims=True)
    cx = jnp.sum(u * sm, axis=0, keepdims=True)
    z = jnp.zeros_like(su)
    st = jnp.concatenate([su, su2, sv, sv2, cx, z, z, z], axis=0)

    @pl.when(t == 0)
    def _():
        st_ref[0] = st

    @pl.when(t > 0)
    def _():
        st_ref[0] = st_ref[0] + st


def _red_call(g, u):
    B, N, Cw = u.shape
    TB = 512
    NT = N // TB
    grid = (B, NT)
    return pl.pallas_call(
        functools.partial(_red_body, Cw),
        grid=grid,
        in_specs=[
            pl.BlockSpec((KNN, TB, Cw), lambda b, t: (0, b * NT + t, 0)),
            pl.BlockSpec((1, TB, Cw), lambda b, t: (b, t, 0)),
        ],
        out_specs=[
            pl.BlockSpec((1, TB, Cw), lambda b, t: (b, t, 0)),
            pl.BlockSpec((1, 8, Cw), lambda b, t: (b, 0, 0)),
        ],
        out_shape=[
            jax.ShapeDtypeStruct((B, N, Cw), jnp.float32),
            jax.ShapeDtypeStruct((B, 8, Cw), jnp.float32),
        ],
    )(g, u)


def _edge_stats(st_ref, n_total):
    st = st_ref[0]
    su, su2, sv, sv2, cx = (st[0:1], st[1:2], st[2:3], st[3:4], st[4:5])
    inv = 1.0 / (n_total * KNN)
    mean = (KNN * su + sv) * inv
    ex2 = (KNN * su2 + 2.0 * cx + sv2) * inv
    var = ex2 - mean * mean
    return mean, lax.rsqrt(var + EPS)


def _norm_uv_body(n_total, pre_ref, st_ref, w_ref, x_ref, u_ref, v_ref):
    mean, rstd = _edge_stats(st_ref, n_total)
    x = _lrelu((pre_ref[0] - mean) * rstd)
    x_ref[0] = x
    w = w_ref[...]
    ci = w.shape[1] // 2
    wa = w[:, :ci] - w[:, ci:]
    wb = w[:, ci:]
    u_ref[0] = lax.dot_general(x, wa, CT11,
                               preferred_element_type=jnp.float32, precision=HI)
    v_ref[...] = lax.dot_general(x, wb, CT11,
                                 preferred_element_type=jnp.float32, precision=HI)


def _norm_uv_call(pre, st, w):
    B, N, Cin = pre.shape
    Cout = w.shape[0]
    TB = 512
    NT = N // TB
    grid = (B, NT)
    return pl.pallas_call(
        functools.partial(_norm_uv_body, N),
        grid=grid,
        in_specs=[
            pl.BlockSpec((1, TB, Cin), lambda b, t: (b, t, 0)),
            pl.BlockSpec((1, 8, Cin), lambda b, t: (b, 0, 0)),
            pl.BlockSpec(w.shape, lambda b, t: (0, 0)),
        ],
        out_specs=[
            pl.BlockSpec((1, TB, Cin), lambda b, t: (b, t, 0)),
            pl.BlockSpec((1, TB, Cout), lambda b, t: (b, t, 0)),
            pl.BlockSpec((TB, Cout), lambda b, t: (b * NT + t, 0)),
        ],
        out_shape=[
            jax.ShapeDtypeStruct((B, N, Cin), jnp.float32),
            jax.ShapeDtypeStruct((B, N, Cout), jnp.float32),
            jax.ShapeDtypeStruct((B * N, Cout), jnp.float32),
        ],
    )(pre, st, w)


def _norm_conv3_body(n_total, c, pre_ref, st_ref, ft_ref, x1_ref, w_ref,
                     y_ref, st3_ref):
    t = pl.program_id(1)
    mean, rstd = _edge_stats(st_ref, n_total)
    x2 = _lrelu((pre_ref[0] - mean) * rstd)
    w = w_ref[...]
    y = lax.dot_general(ft_ref[0], w[:, :c], CT11,
                        preferred_element_type=jnp.float32, precision=HI)
    y = y + lax.dot_general(x1_ref[0], w[:, c:2 * c], CT11,
                            preferred_element_type=jnp.float32, precision=HI)
    y = y + lax.dot_general(x2, w[:, 2 * c:], CT11,
                            preferred_element_type=jnp.float32, precision=HI)
    y_ref[0] = y
    sy = jnp.sum(y, axis=0, keepdims=True)
    sy2 = jnp.sum(y * y, axis=0, keepdims=True)
    z = jnp.zeros_like(sy)
    st = jnp.concatenate([sy, sy2, z, z, z, z, z, z], axis=0)

    @pl.when(t == 0)
    def _():
        st3_ref[0] = st

    @pl.when(t > 0)
    def _():
        st3_ref[0] = st3_ref[0] + st


def _norm_conv3_call(pre2, st2, ftT, x1, w3):
    B, N, C2 = pre2.shape
    C = ftT.shape[2]
    TB = 512
    grid = (B, N // TB)
    return pl.pallas_call(
        functools.partial(_norm_conv3_body, N, C),
        grid=grid,
        in_specs=[
            pl.BlockSpec((1, TB, C2), lambda b, t: (b, t, 0)),
            pl.BlockSpec((1, 8, C2), lambda b, t: (b, 0, 0)),
            pl.BlockSpec((1, TB, C), lambda b, t: (b, t, 0)),
            pl.BlockSpec((1, TB, C), lambda b, t: (b, t, 0)),
            pl.BlockSpec(w3.shape, lambda b, t: (0, 0)),
        ],
        out_specs=[
            pl.BlockSpec((1, TB, C), lambda b, t: (b, t, 0)),
            pl.BlockSpec((1, 8, C), lambda b, t: (b, 0, 0)),
        ],
        out_shape=[
            jax.ShapeDtypeStruct((B, N, C), jnp.float32),
            jax.ShapeDtypeStruct((B, 8, C), jnp.float32),
        ],
    )(pre2, st2, ftT, x1, w3)


def _final_body(n_total, y_ref, st_ref, o_ref):
    st = st_ref[0]
    mean = st[0:1] / n_total
    var = st[1:2] / n_total - mean * mean
    o_ref[0] = _lrelu((y_ref[0] - mean) * lax.rsqrt(var + EPS))


def _final_call(y, st):
    B, N, C = y.shape
    TB = 512
    grid = (B, N // TB)
    return pl.pallas_call(
        functools.partial(_final_body, N),
        grid=grid,
        in_specs=[
            pl.BlockSpec((1, TB, C), lambda b, t: (b, t, 0)),
            pl.BlockSpec((1, 8, C), lambda b, t: (b, 0, 0)),
        ],
        out_specs=pl.BlockSpec((1, TB, C), lambda b, t: (b, t, 0)),
        out_shape=jax.ShapeDtypeStruct((B, N, C), jnp.float32),
    )(y, st)


def kernel(coords, features, W1, W2, W3):
    B, C, N = features.shape
    pts = jnp.transpose(coords, (0, 2, 1))
    pts8 = jnp.concatenate(
        [pts, jnp.zeros((B, N, 5), jnp.float32)], axis=2)
    cf8 = jnp.concatenate(
        [coords, jnp.zeros((B, 5, N), jnp.float32)], axis=1)
    ftT = jnp.transpose(features, (0, 2, 1))

    idx16, u1, v1 = _topk_call(pts8, cf8, ftT, W1)
    gidx = jnp.transpose(idx16[:, :, :KNN], (2, 0, 1)).reshape(-1)
    g1 = _sc_gather(v1, gidx).reshape(KNN, B * N, C)
    pre1, st1 = _red_call(g1, u1)
    x1, u2, v2 = _norm_uv_call(pre1, st1, W2)
    g2 = _sc_gather(v2, gidx).reshape(KNN, B * N, 2 * C)
    pre2, st2 = _red_call(g2, u2)
    y3, st3 = _norm_conv3_call(pre2, st2, ftT, x1, W3)
    out = _final_call(y3, st3)
    return jnp.transpose(out, (0, 2, 1))


# speedup vs baseline: 1.2182x; 1.0126x over previous
import functools

import jax
import jax.numpy as jnp
from jax import lax
from jax.experimental import pallas as pl
from jax.experimental.pallas import tpu as pltpu
from jax.experimental.pallas import tpu_sc as plsc

KNN = 10
EPS = 1e-5
BIGF = 3.0e38
CT11 = (((1,), (1,)), ((), ()))
HI = jax.lax.Precision.HIGHEST


def _lrelu(x):
    return jnp.where(x > 0, x, 0.2 * x)


def _topk_body(n_total, tq, pts_ref, cf_ref, x_ref, w_ref,
               idx_ref, u_ref, v_ref):
    b = pl.program_id(0)
    t = pl.program_id(1)
    x = x_ref[0]
    w = w_ref[...]
    ci = w.shape[1] // 2
    wa = w[:, :ci] - w[:, ci:]
    wb = w[:, ci:]
    u_ref[0] = lax.dot_general(x, wa, CT11,
                               preferred_element_type=jnp.float32, precision=HI)
    v_ref[...] = lax.dot_general(x, wb, CT11,
                                 preferred_element_type=jnp.float32, precision=HI)
    a = pts_ref[0]
    c = cf_ref[0]
    g = lax.dot_general(a, c, (((1,), (0,)), ((), ())),
                        preferred_element_type=jnp.float32,
                        precision=lax.Precision.DEFAULT)
    n2r = jnp.sum(a * a, axis=1, keepdims=True)
    n2c = jnp.sum(c * c, axis=0, keepdims=True)
    d = jnp.maximum((-2.0 * g + n2r) + n2c, 1e-12)
    colid = lax.broadcasted_iota(jnp.int32, d.shape, 1)
    base = b * n_total
    for kk in range(KNN + 1):
        rowmin = jnp.min(d, axis=1, keepdims=True)
        amin = jnp.min(jnp.where(d == rowmin, colid, n_total),
                       axis=1, keepdims=True)
        if kk > 0:
            idx_ref[0, :, kk - 1:kk] = amin + base
        if kk <= KNN - 1:
            d = jnp.where(colid == amin, BIGF, d)


def _topk_call(pts8, cf8, ftT, w1):
    B, N, _ = pts8.shape
    Cin = ftT.shape[2]
    Cout = w1.shape[0]
    TQ = 256
    NT = N // TQ
    grid = (B, NT)
    return pl.pallas_call(
        functools.partial(_topk_body, N, TQ),
        grid=grid,
        in_specs=[
            pl.BlockSpec((1, TQ, 8), lambda b, t: (b, t, 0)),
            pl.BlockSpec((1, 8, N), lambda b, t: (b, 0, 0)),
            pl.BlockSpec((1, TQ, Cin), lambda b, t: (b, t, 0)),
            pl.BlockSpec(w1.shape, lambda b, t: (0, 0)),
        ],
        out_specs=[
            pl.BlockSpec((1, TQ, 16), lambda b, t: (b, t, 0)),
            pl.BlockSpec((1, TQ, Cout), lambda b, t: (b, t, 0)),
            pl.BlockSpec((TQ, Cout), lambda b, t: (b * NT + t, 0)),
        ],
        out_shape=[
            jax.ShapeDtypeStruct((B, N, 16), jnp.int32),
            jax.ShapeDtypeStruct((B, N, Cout), jnp.float32),
            jax.ShapeDtypeStruct((B * N, Cout), jnp.float32),
        ],
    )(pts8, cf8, ftT, w1)


def _sc_gather(table, gidx):
    R, D = table.shape
    M = gidx.shape[0]
    NW = 32
    b_per_w = M // NW
    CH = 32768 // D
    n_pairs = b_per_w // (2 * CH)
    mesh = plsc.VectorSubcoreMesh(core_axis_name="c", subcore_axis_name="s")

    @functools.partial(
        pl.kernel, mesh=mesh,
        out_type=jax.ShapeDtypeStruct((M, D), jnp.float32),
        scratch_types=[
            pltpu.VMEM((b_per_w,), jnp.int32),
            pltpu.VMEM((CH, D), jnp.float32),
            pltpu.VMEM((CH, D), jnp.float32),
            pltpu.SemaphoreType.DMA,
            pltpu.SemaphoreType.DMA,
            pltpu.SemaphoreType.DMA,
        ],
    )
    def k(table_hbm, idx_hbm, out_hbm, idx_all, r0, r1, sg, s0, s1):
        wid = lax.axis_index("s") * 2 + lax.axis_index("c")
        base = wid * b_per_w
        pltpu.sync_copy(idx_hbm.at[pl.ds(base, b_per_w)], idx_all)

        @pl.loop(0, n_pairs)
        def _(t):
            o0 = 2 * t * CH
            o1 = o0 + CH

            @pl.when(t > 0)
            def _():
                pltpu.make_async_copy(
                    r0, out_hbm.at[pl.ds(base + o0 - 2 * CH, CH)], s0).wait()

            pltpu.async_copy(
                table_hbm.at[idx_all.at[pl.ds(o0, CH)]], r0, sg).wait()
            pltpu.async_copy(r0, out_hbm.at[pl.ds(base + o0, CH)], s0)

            @pl.when(t > 0)
            def _():
                pltpu.make_async_copy(
                    r1, out_hbm.at[pl.ds(base + o1 - 2 * CH, CH)], s1).wait()

            pltpu.async_copy(
                table_hbm.at[idx_all.at[pl.ds(o1, CH)]], r1, sg).wait()
            pltpu.async_copy(r1, out_hbm.at[pl.ds(base + o1, CH)], s1)

        last0 = b_per_w - 2 * CH
        pltpu.make_async_copy(
            r0, out_hbm.at[pl.ds(base + last0, CH)], s0).wait()
        pltpu.make_async_copy(
            r1, out_hbm.at[pl.ds(base + last0 + CH, CH)], s1).wait()

    return k(table, gidx)


def _k_reduce(g_ref):
    g0 = g_ref[0]
    mx = g0
    sm = g0
    s2 = g0 * g0
    for kk in range(1, KNN):
        gk = g_ref[kk]
        mx = jnp.maximum(mx, gk)
        sm = sm + gk
        s2 = s2 + gk * gk
    return mx, sm, s2


def _edge_stat_rows(u, sm, s2):
    su = jnp.sum(u, axis=0, keepdims=True)
    su2 = jnp.sum(u * u, axis=0, keepdims=True)
    sv = jnp.sum(sm, axis=0, keepdims=True)
    sv2 = jnp.sum(s2, axis=0, keepdims=True)
    cx = jnp.sum(u * sm, axis=0, keepdims=True)
    z = jnp.zeros_like(su)
    return jnp.concatenate([su, su2, sv, sv2, cx, z, z, z], axis=0)


def _edge_stats(st, n_total):
    su, su2, sv, sv2, cx = (st[0:1], st[1:2], st[2:3], st[3:4], st[4:5])
    inv = 1.0 / (n_total * KNN)
    mean = (KNN * su + sv) * inv
    ex2 = (KNN * su2 + 2.0 * cx + sv2) * inv
    var = ex2 - mean * mean
    return mean, lax.rsqrt(var + EPS)


def _l1_body(nt, tb, n_total, g_ref, u_ref, w_ref, x_ref, u2_ref, v2_ref,
             pre_s, st_s):
    t = pl.program_id(1)

    @pl.when(t < nt)
    def _():
        mx, sm, s2 = _k_reduce(g_ref)
        u = u_ref[0]
        pre_s[pl.ds(t * tb, tb), :] = u + mx
        st = _edge_stat_rows(u, sm, s2)

        @pl.when(t == 0)
        def _():
            st_s[...] = st

        @pl.when(t > 0)
        def _():
            st_s[...] = st_s[...] + st

    @pl.when(t >= nt)
    def _():
        tt = t - nt
        mean, rstd = _edge_stats(st_s[...], n_total)
        x = _lrelu((pre_s[pl.ds(tt * tb, tb), :] - mean) * rstd)
        x_ref[0] = x
        w = w_ref[...]
        ci = w.shape[1] // 2
        wa = w[:, :ci] - w[:, ci:]
        wb = w[:, ci:]
        u2_ref[0] = lax.dot_general(
            x, wa, CT11, preferred_element_type=jnp.float32, precision=HI)
        v2_ref[...] = lax.dot_general(
            x, wb, CT11, preferred_element_type=jnp.float32, precision=HI)


def _l1_call(g, u, w):
    B, N, Cw = u.shape
    Cout = w.shape[0]
    TB = 512
    NT = N // TB
    grid = (B, 2 * NT)
    return pl.pallas_call(
        functools.partial(_l1_body, NT, TB, N),
        grid=grid,
        in_specs=[
            pl.BlockSpec((KNN, TB, Cw),
                         lambda b, t: (0, b * NT + jnp.where(t < NT, t, 0), 0)),
            pl.BlockSpec((1, TB, Cw),
                         lambda b, t: (b, jnp.where(t < NT, t, 0), 0)),
            pl.BlockSpec(w.shape, lambda b, t: (0, 0)),
        ],
        out_specs=[
            pl.BlockSpec((1, TB, Cw),
                         lambda b, t: (b, jnp.where(t < NT, 0, t - NT), 0)),
            pl.BlockSpec((1, TB, Cout),
                         lambda b, t: (b, jnp.where(t < NT, 0, t - NT), 0)),
            pl.BlockSpec((TB, Cout),
                         lambda b, t: (b * NT + jnp.where(t < NT, 0, t - NT),
                                       0)),
        ],
        out_shape=[
            jax.ShapeDtypeStruct((B, N, Cw), jnp.float32),
            jax.ShapeDtypeStruct((B, N, Cout), jnp.float32),
            jax.ShapeDtypeStruct((B * N, Cout), jnp.float32),
        ],
        scratch_shapes=[
            pltpu.VMEM((N, Cw), jnp.float32),
            pltpu.VMEM((8, Cw), jnp.float32),
        ],
    )(g, u, w)


def _l2_body(nt, tb, n_total, c, g_ref, u_ref, ft_ref, x1_ref, w_ref,
             out_ref, pre_s, y_s, st2_s, st3_s):
    t = pl.program_id(1)

    @pl.when(t < nt)
    def _():
        mx, sm, s2 = _k_reduce(g_ref)
        u = u_ref[0]
        pre_s[pl.ds(t * tb, tb), :] = u + mx
        st = _edge_stat_rows(u, sm, s2)

        @pl.when(t == 0)
        def _():
            st2_s[...] = st

        @pl.when(t > 0)
        def _():
            st2_s[...] = st2_s[...] + st

    @pl.when(jnp.logical_and(t >= nt, t < 2 * nt))
    def _():
        tt = t - nt
        mean, rstd = _edge_stats(st2_s[...], n_total)
        x2 = _lrelu((pre_s[pl.ds(tt * tb, tb), :] - mean) * rstd)
        w = w_ref[...]
        y = lax.dot_general(ft_ref[0], w[:, :c], CT11,
                            preferred_element_type=jnp.float32, precision=HI)
        y = y + lax.dot_general(x1_ref[0], w[:, c:2 * c], CT11,
                                preferred_element_type=jnp.float32,
                                precision=HI)
        y = y + lax.dot_general(x2, w[:, 2 * c:], CT11,
                                preferred_element_type=jnp.float32,
                                precision=HI)
        y_s[pl.ds(tt * tb, tb), :] = y
        sy = jnp.sum(y, axis=0, keepdims=True)
        sy2 = jnp.sum(y * y, axis=0, keepdims=True)
        z = jnp.zeros_like(sy)
        st = jnp.concatenate([sy, sy2, z, z, z, z, z, z], axis=0)

        @pl.when(tt == 0)
        def _():
            st3_s[...] = st

        @pl.when(tt > 0)
        def _():
            st3_s[...] = st3_s[...] + st

    @pl.when(t >= 2 * nt)
    def _():
        tt = t - 2 * nt
        st = st3_s[...]
        mean = st[0:1] / n_total
        var = st[1:2] / n_total - mean * mean
        out_ref[0] = _lrelu(
            (y_s[pl.ds(tt * tb, tb), :] - mean) * lax.rsqrt(var + EPS))


def _l2_call(g, u2, ftT, x1, w3):
    B, N, C2 = u2.shape
    C = ftT.shape[2]
    TB = 512
    NT = N // TB
    grid = (B, 3 * NT)
    return pl.pallas_call(
        functools.partial(_l2_body, NT, TB, N, C),
        grid=grid,
        in_specs=[
            pl.BlockSpec((KNN, TB, C2),
                         lambda b, t: (0, b * NT + jnp.where(t < NT, t, 0), 0)),
            pl.BlockSpec((1, TB, C2),
                         lambda b, t: (b, jnp.where(t < NT, t, 0), 0)),
            pl.BlockSpec((1, TB, C),
                         lambda b, t: (b, jnp.where(
                             jnp.logical_and(t >= NT, t < 2 * NT),
                             t - NT, 0), 0)),
            pl.BlockSpec((1, TB, C),
                         lambda b, t: (b, jnp.where(
                             jnp.logical_and(t >= NT, t < 2 * NT),
                             t - NT, 0), 0)),
            pl.BlockSpec(w3.shape, lambda b, t: (0, 0)),
        ],
        out_specs=pl.BlockSpec(
            (1, TB, C),
            lambda b, t: (b, jnp.where(t < 2 * NT, 0, t - 2 * NT), 0)),
        out_shape=jax.ShapeDtypeStruct((B, N, C), jnp.float32),
        scratch_shapes=[
            pltpu.VMEM((N, C2), jnp.float32),
            pltpu.VMEM((N, C), jnp.float32),
            pltpu.VMEM((8, C2), jnp.float32),
            pltpu.VMEM((8, C), jnp.float32),
        ],
    )(g, u2, ftT, x1, w3)


def kernel(coords, features, W1, W2, W3):
    B, C, N = features.shape
    pts = jnp.transpose(coords, (0, 2, 1))
    pts8 = jnp.concatenate(
        [pts, jnp.zeros((B, N, 5), jnp.float32)], axis=2)
    cf8 = jnp.concatenate(
        [coords, jnp.zeros((B, 5, N), jnp.float32)], axis=1)
    ftT = jnp.transpose(features, (0, 2, 1))

    idx16, u1, v1 = _topk_call(pts8, cf8, ftT, W1)
    gidx = jnp.transpose(idx16[:, :, :KNN], (2, 0, 1)).reshape(-1)
    g1 = _sc_gather(v1, gidx).reshape(KNN, B * N, C)
    x1, u2, v2 = _l1_call(g1, u1, W2)
    g2 = _sc_gather(v2, gidx).reshape(KNN, B * N, 2 * C)
    out = _l2_call(g2, u2, ftT, x1, W3)
    return jnp.transpose(out, (0, 2, 1))


# speedup vs baseline: 1.2723x; 1.0444x over previous
import functools

import jax
import jax.numpy as jnp
from jax import lax
from jax.experimental import pallas as pl
from jax.experimental.pallas import tpu as pltpu
from jax.experimental.pallas import tpu_sc as plsc

KNN = 10
EPS = 1e-5
BIGF = 3.0e38
CT11 = (((1,), (1,)), ((), ()))
HI = jax.lax.Precision.HIGHEST


def _lrelu(x):
    return jnp.where(x > 0, x, 0.2 * x)


def _topk_body(n_total, tq, pts_ref, cf_ref, x_ref, w_ref,
               idx_ref, u_ref, v_ref):
    b = pl.program_id(0)
    t = pl.program_id(1)
    x = x_ref[0]
    w = w_ref[...]
    ci = w.shape[1] // 2
    wa = w[:, :ci] - w[:, ci:]
    wb = w[:, ci:]
    u_ref[0] = lax.dot_general(x, wa, CT11,
                               preferred_element_type=jnp.float32, precision=HI)
    v_ref[...] = lax.dot_general(x, wb, CT11,
                                 preferred_element_type=jnp.float32, precision=HI)
    a = pts_ref[0]
    c = cf_ref[0]
    g = lax.dot_general(a, c, (((1,), (0,)), ((), ())),
                        preferred_element_type=jnp.float32,
                        precision=lax.Precision.DEFAULT)
    n2r = jnp.sum(a * a, axis=1, keepdims=True)
    n2c = jnp.sum(c * c, axis=0, keepdims=True)
    d = jnp.maximum((-2.0 * g + n2r) + n2c, 1e-12)
    colid = lax.broadcasted_iota(jnp.int32, d.shape, 1)
    base = b * n_total
    for kk in range(KNN + 1):
        rowmin = jnp.min(d, axis=1, keepdims=True)
        amin = jnp.min(jnp.where(d == rowmin, colid, n_total),
                       axis=1, keepdims=True)
        if kk > 0:
            idx_ref[0, :, kk - 1:kk] = amin + base
        if kk <= KNN - 1:
            d = jnp.where(colid == amin, BIGF, d)


def _topk_call(pts8, cf8, ftT, w1):
    B, N, _ = pts8.shape
    Cin = ftT.shape[2]
    Cout = w1.shape[0]
    TQ = 512
    NT = N // TQ
    grid = (B, NT)
    return pl.pallas_call(
        functools.partial(_topk_body, N, TQ),
        grid=grid,
        in_specs=[
            pl.BlockSpec((1, TQ, 8), lambda b, t: (b, t, 0)),
            pl.BlockSpec((1, 8, N), lambda b, t: (b, 0, 0)),
            pl.BlockSpec((1, TQ, Cin), lambda b, t: (b, t, 0)),
            pl.BlockSpec(w1.shape, lambda b, t: (0, 0)),
        ],
        out_specs=[
            pl.BlockSpec((1, TQ, 16), lambda b, t: (b, t, 0)),
            pl.BlockSpec((1, TQ, Cout), lambda b, t: (b, t, 0)),
            pl.BlockSpec((TQ, Cout), lambda b, t: (b * NT + t, 0)),
        ],
        out_shape=[
            jax.ShapeDtypeStruct((B, N, 16), jnp.int32),
            jax.ShapeDtypeStruct((B, N, Cout), jnp.float32),
            jax.ShapeDtypeStruct((B * N, Cout), jnp.float32),
        ],
    )(pts8, cf8, ftT, w1)


def _sc_gather(table, gidx):
    R, D = table.shape
    M = gidx.shape[0]
    NW = 32
    b_per_w = M // NW
    CH = 32768 // D
    n_pairs = b_per_w // (2 * CH)
    mesh = plsc.VectorSubcoreMesh(core_axis_name="c", subcore_axis_name="s")

    @functools.partial(
        pl.kernel, mesh=mesh,
        out_type=jax.ShapeDtypeStruct((M, D), jnp.float32),
        scratch_types=[
            pltpu.VMEM((b_per_w,), jnp.int32),
            pltpu.VMEM((CH, D), jnp.float32),
            pltpu.VMEM((CH, D), jnp.float32),
            pltpu.SemaphoreType.DMA,
            pltpu.SemaphoreType.DMA,
            pltpu.SemaphoreType.DMA,
        ],
    )
    def k(table_hbm, idx_hbm, out_hbm, idx_all, r0, r1, sg, s0, s1):
        wid = lax.axis_index("s") * 2 + lax.axis_index("c")
        base = wid * b_per_w
        pltpu.sync_copy(idx_hbm.at[pl.ds(base, b_per_w)], idx_all)

        @pl.loop(0, n_pairs)
        def _(t):
            o0 = 2 * t * CH
            o1 = o0 + CH

            @pl.when(t > 0)
            def _():
                pltpu.make_async_copy(
                    r0, out_hbm.at[pl.ds(base + o0 - 2 * CH, CH)], s0).wait()

            pltpu.async_copy(
                table_hbm.at[idx_all.at[pl.ds(o0, CH)]], r0, sg).wait()
            pltpu.async_copy(r0, out_hbm.at[pl.ds(base + o0, CH)], s0)

            @pl.when(t > 0)
            def _():
                pltpu.make_async_copy(
                    r1, out_hbm.at[pl.ds(base + o1 - 2 * CH, CH)], s1).wait()

            pltpu.async_copy(
                table_hbm.at[idx_all.at[pl.ds(o1, CH)]], r1, sg).wait()
            pltpu.async_copy(r1, out_hbm.at[pl.ds(base + o1, CH)], s1)

        last0 = b_per_w - 2 * CH
        pltpu.make_async_copy(
            r0, out_hbm.at[pl.ds(base + last0, CH)], s0).wait()
        pltpu.make_async_copy(
            r1, out_hbm.at[pl.ds(base + last0 + CH, CH)], s1).wait()

    return k(table, gidx)


def _k_reduce(g_ref):
    g0 = g_ref[0]
    mx = g0
    sm = g0
    s2 = g0 * g0
    for kk in range(1, KNN):
        gk = g_ref[kk]
        mx = jnp.maximum(mx, gk)
        sm = sm + gk
        s2 = s2 + gk * gk
    return mx, sm, s2


def _edge_stat_rows(u, sm, s2):
    su = jnp.sum(u, axis=0, keepdims=True)
    su2 = jnp.sum(u * u, axis=0, keepdims=True)
    sv = jnp.sum(sm, axis=0, keepdims=True)
    sv2 = jnp.sum(s2, axis=0, keepdims=True)
    cx = jnp.sum(u * sm, axis=0, keepdims=True)
    z = jnp.zeros_like(su)
    return jnp.concatenate([su, su2, sv, sv2, cx, z, z, z], axis=0)


def _edge_stats(st, n_total):
    su, su2, sv, sv2, cx = (st[0:1], st[1:2], st[2:3], st[3:4], st[4:5])
    inv = 1.0 / (n_total * KNN)
    mean = (KNN * su + sv) * inv
    ex2 = (KNN * su2 + 2.0 * cx + sv2) * inv
    var = ex2 - mean * mean
    return mean, lax.rsqrt(var + EPS)


def _l1_body(nt, tb, n_total, g_ref, u_ref, w_ref, x_ref, u2_ref, v2_ref,
             pre_s, st_s):
    t = pl.program_id(1)

    @pl.when(t < nt)
    def _():
        mx, sm, s2 = _k_reduce(g_ref)
        u = u_ref[0]
        pre_s[pl.ds(t * tb, tb), :] = u + mx
        st = _edge_stat_rows(u, sm, s2)

        @pl.when(t == 0)
        def _():
            st_s[...] = st

        @pl.when(t > 0)
        def _():
            st_s[...] = st_s[...] + st

    @pl.when(t >= nt)
    def _():
        tt = t - nt
        mean, rstd = _edge_stats(st_s[...], n_total)
        x = _lrelu((pre_s[pl.ds(tt * tb, tb), :] - mean) * rstd)
        x_ref[0] = x
        w = w_ref[...]
        ci = w.shape[1] // 2
        wa = w[:, :ci] - w[:, ci:]
        wb = w[:, ci:]
        u2_ref[0] = lax.dot_general(
            x, wa, CT11, preferred_element_type=jnp.float32, precision=HI)
        v2_ref[...] = lax.dot_general(
            x, wb, CT11, preferred_element_type=jnp.float32, precision=HI)


def _l1_call(g, u, w):
    B, N, Cw = u.shape
    Cout = w.shape[0]
    TB = 512
    NT = N // TB
    grid = (B, 2 * NT)
    return pl.pallas_call(
        functools.partial(_l1_body, NT, TB, N),
        grid=grid,
        in_specs=[
            pl.BlockSpec((KNN, TB, Cw),
                         lambda b, t: (0, b * NT + jnp.where(t < NT, t, 0), 0)),
            pl.BlockSpec((1, TB, Cw),
                         lambda b, t: (b, jnp.where(t < NT, t, 0), 0)),
            pl.BlockSpec(w.shape, lambda b, t: (0, 0)),
        ],
        out_specs=[
            pl.BlockSpec((1, TB, Cw),
                         lambda b, t: (b, jnp.where(t < NT, 0, t - NT), 0)),
            pl.BlockSpec((1, TB, Cout),
                         lambda b, t: (b, jnp.where(t < NT, 0, t - NT), 0)),
            pl.BlockSpec((TB, Cout),
                         lambda b, t: (b * NT + jnp.where(t < NT, 0, t - NT),
                                       0)),
        ],
        out_shape=[
            jax.ShapeDtypeStruct((B, N, Cw), jnp.float32),
            jax.ShapeDtypeStruct((B, N, Cout), jnp.float32),
            jax.ShapeDtypeStruct((B * N, Cout), jnp.float32),
        ],
        scratch_shapes=[
            pltpu.VMEM((N, Cw), jnp.float32),
            pltpu.VMEM((8, Cw), jnp.float32),
        ],
    )(g, u, w)


def _l2_body(nt, tb, n_total, c, g_ref, u_ref, ft_ref, x1_ref, w_ref,
             out_ref, pre_s, y_s, st2_s, st3_s):
    t = pl.program_id(1)

    @pl.when(t < nt)
    def _():
        mx, sm, s2 = _k_reduce(g_ref)
        u = u_ref[0]
        pre_s[pl.ds(t * tb, tb), :] = u + mx
        st = _edge_stat_rows(u, sm, s2)

        @pl.when(t == 0)
        def _():
            st2_s[...] = st

        @pl.when(t > 0)
        def _():
            st2_s[...] = st2_s[...] + st

    @pl.when(jnp.logical_and(t >= nt, t < 2 * nt))
    def _():
        tt = t - nt
        mean, rstd = _edge_stats(st2_s[...], n_total)
        x2 = _lrelu((pre_s[pl.ds(tt * tb, tb), :] - mean) * rstd)
        w = w_ref[...]
        y = lax.dot_general(ft_ref[0], w[:, :c], CT11,
                            preferred_element_type=jnp.float32, precision=HI)
        y = y + lax.dot_general(x1_ref[0], w[:, c:2 * c], CT11,
                                preferred_element_type=jnp.float32,
                                precision=HI)
        y = y + lax.dot_general(x2, w[:, 2 * c:], CT11,
                                preferred_element_type=jnp.float32,
                                precision=HI)
        y_s[pl.ds(tt * tb, tb), :] = y
        sy = jnp.sum(y, axis=0, keepdims=True)
        sy2 = jnp.sum(y * y, axis=0, keepdims=True)
        z = jnp.zeros_like(sy)
        st = jnp.concatenate([sy, sy2, z, z, z, z, z, z], axis=0)

        @pl.when(tt == 0)
        def _():
            st3_s[...] = st

        @pl.when(tt > 0)
        def _():
            st3_s[...] = st3_s[...] + st

    @pl.when(t >= 2 * nt)
    def _():
        tt = t - 2 * nt
        st = st3_s[...]
        mean = st[0:1] / n_total
        var = st[1:2] / n_total - mean * mean
        out_ref[0] = _lrelu(
            (y_s[pl.ds(tt * tb, tb), :] - mean) * lax.rsqrt(var + EPS))


def _l2_call(g, u2, ftT, x1, w3):
    B, N, C2 = u2.shape
    C = ftT.shape[2]
    TB = 512
    NT = N // TB
    grid = (B, 3 * NT)
    return pl.pallas_call(
        functools.partial(_l2_body, NT, TB, N, C),
        grid=grid,
        in_specs=[
            pl.BlockSpec((KNN, TB, C2),
                         lambda b, t: (0, b * NT + jnp.where(t < NT, t, 0), 0)),
            pl.BlockSpec((1, TB, C2),
                         lambda b, t: (b, jnp.where(t < NT, t, 0), 0)),
            pl.BlockSpec((1, TB, C),
                         lambda b, t: (b, jnp.where(
                             jnp.logical_and(t >= NT, t < 2 * NT),
                             t - NT, 0), 0)),
            pl.BlockSpec((1, TB, C),
                         lambda b, t: (b, jnp.where(
                             jnp.logical_and(t >= NT, t < 2 * NT),
                             t - NT, 0), 0)),
            pl.BlockSpec(w3.shape, lambda b, t: (0, 0)),
        ],
        out_specs=pl.BlockSpec(
            (1, TB, C),
            lambda b, t: (b, jnp.where(t < 2 * NT, 0, t - 2 * NT), 0)),
        out_shape=jax.ShapeDtypeStruct((B, N, C), jnp.float32),
        scratch_shapes=[
            pltpu.VMEM((N, C2), jnp.float32),
            pltpu.VMEM((N, C), jnp.float32),
            pltpu.VMEM((8, C2), jnp.float32),
            pltpu.VMEM((8, C), jnp.float32),
        ],
    )(g, u2, ftT, x1, w3)


def kernel(coords, features, W1, W2, W3):
    B, C, N = features.shape
    pts = jnp.transpose(coords, (0, 2, 1))
    pts8 = jnp.concatenate(
        [pts, jnp.zeros((B, N, 5), jnp.float32)], axis=2)
    cf8 = jnp.concatenate(
        [coords, jnp.zeros((B, 5, N), jnp.float32)], axis=1)
    ftT = jnp.transpose(features, (0, 2, 1))

    idx16, u1, v1 = _topk_call(pts8, cf8, ftT, W1)
    gidx = jnp.transpose(idx16[:, :, :KNN], (2, 0, 1)).reshape(-1)
    g1 = _sc_gather(v1, gidx).reshape(KNN, B * N, C)
    x1, u2, v2 = _l1_call(g1, u1, W2)
    g2 = _sc_gather(v2, gidx).reshape(KNN, B * N, 2 * C)
    out = _l2_call(g2, u2, ftT, x1, W3)
    return jnp.transpose(out, (0, 2, 1))
